# baseline (speedup 1.0000x reference)
import jax
import jax.numpy as jnp
from jax import lax
from jax.experimental import pallas as pl
from jax.experimental.pallas import tpu as pltpu

_RB = 4096
_WR = 2048


def _flatroll(x, s):
    sl = jnp.mod(s, 128)
    sr = s // 128
    xr = pltpu.roll(x, sl, axis=1)
    y0 = pltpu.roll(xr, sr, axis=0)
    y1 = pltpu.roll(xr, sr + 1, axis=0)
    col = lax.broadcasted_iota(jnp.int32, x.shape, 1)
    return jnp.where(col < sl, y1, y0)


def _emb_body(s_ref, emb, qe, se, ne, er_s):
    eb = _RB * 128
    ce = pl.num_programs(0) * eb
    wl = _WR * 128
    w0 = s_ref[0]
    g = pl.program_id(0)

    @pl.when(g == 0)
    def _():
        er_s[...] = _flatroll(emb[...], jnp.mod(w0, wl))

    se[...] = qe[...]
    t0 = jnp.mod(g * eb - w0, ce)
    er = er_s[...]
    for k in range(_RB // _WR):
        fi = (lax.broadcasted_iota(jnp.int32, (_WR, 128), 0) * 128
              + lax.broadcasted_iota(jnp.int32, (_WR, 128), 1))
        tt = t0 + k * (_WR * 128) + fi
        tt = jnp.where(tt >= ce, tt - ce, tt)
        mask = tt < wl
        ne[k * _WR:(k + 1) * _WR, :] = jnp.where(mask, er, qe[k * _WR:(k + 1) * _WR, :])


def _ids_body(s_ref, ilo, ihi, qlo, qhi, slo, shi, nlo, nhi, rlo_s, rhi_s):
    rows, cols = qlo.shape
    wrows = ilo.shape[0]
    ce = rows * cols
    wl = wrows * cols
    w0 = s_ref[0]

    rlo_s[...] = _flatroll(ilo[...], jnp.mod(w0, wl))
    rhi_s[...] = _flatroll(ihi[...], jnp.mod(w0, wl))

    slo[...] = qlo[...]
    shi[...] = qhi[...]
    rlo = rlo_s[...]
    rhi = rhi_s[...]
    for k in range(rows // wrows):
        fi = (lax.broadcasted_iota(jnp.int32, (wrows, cols), 0) * cols
              + lax.broadcasted_iota(jnp.int32, (wrows, cols), 1))
        tt = k * wl + fi - w0
        tt = jnp.where(tt < 0, tt + ce, tt)
        mask = tt < wl
        nlo[k * wrows:(k + 1) * wrows, :] = jnp.where(mask, rlo, qlo[k * wrows:(k + 1) * wrows, :])
        nhi[k * wrows:(k + 1) * wrows, :] = jnp.where(mask, rhi, qhi[k * wrows:(k + 1) * wrows, :])


def kernel(embeddings, item_ids, queue_embeddings, queue_item_ids, ptr):
    C, D = queue_embeddings.shape
    B = embeddings.shape[0]
    p = jnp.asarray(jnp.mod(ptr, C), jnp.int32)

    CE = C * D
    G = CE // (_RB * 128)
    emb2 = embeddings.reshape(_WR, 128)
    qe2 = queue_embeddings.reshape(CE // 128, 128)
    scal = jnp.stack([D * p, jnp.int32(0)])

    se2, ne2 = pl.pallas_call(
        _emb_body,
        grid_spec=pltpu.PrefetchScalarGridSpec(
            num_scalar_prefetch=1,
            grid=(G,),
            in_specs=[
                pl.BlockSpec((_WR, 128), lambda g, pr: (jnp.int32(0), jnp.int32(0))),
                pl.BlockSpec((_RB, 128), lambda g, pr: (g, jnp.int32(0))),
            ],
            out_specs=[
                pl.BlockSpec((_RB, 128), lambda g, pr: (g, jnp.int32(0))),
                pl.BlockSpec((_RB, 128), lambda g, pr: (g, jnp.int32(0))),
            ],
            scratch_shapes=[pltpu.VMEM((_WR, 128), jnp.float32)],
        ),
        out_shape=[
            jax.ShapeDtypeStruct((CE // 128, 128), jnp.float32),
            jax.ShapeDtypeStruct((CE // 128, 128), jnp.float32),
        ],
    )(scal, emb2, qe2)

    ilo = item_ids.astype(jnp.int32).reshape(B // 128, 128)
    ihi = jnp.right_shift(item_ids, 32).astype(jnp.int32).reshape(B // 128, 128)
    qlo = queue_item_ids.astype(jnp.int32).reshape(C // 128, 128)
    qhi = jnp.right_shift(queue_item_ids, 32).astype(jnp.int32).reshape(C // 128, 128)
    scal2 = jnp.stack([p, jnp.int32(0)])

    win_spec = pl.BlockSpec((B // 128, 128), lambda g, pr: (jnp.int32(0), jnp.int32(0)))
    full_spec = pl.BlockSpec((C // 128, 128), lambda g, pr: (jnp.int32(0), jnp.int32(0)))
    slo, shi, nlo, nhi = pl.pallas_call(
        _ids_body,
        grid_spec=pltpu.PrefetchScalarGridSpec(
            num_scalar_prefetch=1,
            grid=(1,),
            in_specs=[win_spec, win_spec, full_spec, full_spec],
            out_specs=[full_spec, full_spec, full_spec, full_spec],
            scratch_shapes=[
                pltpu.VMEM((B // 128, 128), jnp.int32),
                pltpu.VMEM((B // 128, 128), jnp.int32),
            ],
        ),
        out_shape=[
            jax.ShapeDtypeStruct((C // 128, 128), jnp.int32),
            jax.ShapeDtypeStruct((C // 128, 128), jnp.int32),
            jax.ShapeDtypeStruct((C // 128, 128), jnp.int32),
            jax.ShapeDtypeStruct((C // 128, 128), jnp.int32),
        ],
    )(scal2, ilo, ihi, qlo, qhi)

    def _to64(hi, lo):
        return (jnp.left_shift(hi.reshape(C).astype(jnp.int64), 32)
                | (lo.reshape(C).astype(jnp.int64) & jnp.int64(0xFFFFFFFF)))

    se = se2.reshape(C, D)
    ne = ne2.reshape(C, D)
    return (se, _to64(shi, slo), ne, _to64(nhi, nlo))


# speedup vs baseline: 1.0149x; 1.0149x over previous
import jax
import jax.numpy as jnp
from jax import lax
from jax.experimental import pallas as pl
from jax.experimental.pallas import tpu as pltpu

_RB = 16384
_WR = 2048


def _flatroll(x, s):
    sl = jnp.mod(s, 128)
    sr = s // 128
    xr = pltpu.roll(x, sl, axis=1)
    y0 = pltpu.roll(xr, sr, axis=0)
    y1 = pltpu.roll(xr, sr + 1, axis=0)
    col = lax.broadcasted_iota(jnp.int32, x.shape, 1)
    return jnp.where(col < sl, y1, y0)


def _emb_body(s_ref, emb, qe, se, ne, er_s):
    eb = _RB * 128
    ce = pl.num_programs(0) * eb
    wl = _WR * 128
    w0 = s_ref[0]
    g = pl.program_id(0)

    @pl.when(g == 0)
    def _():
        er_s[...] = _flatroll(emb[...], jnp.mod(w0, wl))

    se[...] = qe[...]
    t0 = jnp.mod(g * eb - w0, ce)
    er = er_s[...]
    for k in range(_RB // _WR):
        fi = (lax.broadcasted_iota(jnp.int32, (_WR, 128), 0) * 128
              + lax.broadcasted_iota(jnp.int32, (_WR, 128), 1))
        tt = t0 + k * (_WR * 128) + fi
        tt = jnp.where(tt >= ce, tt - ce, tt)
        mask = tt < wl
        ne[k * _WR:(k + 1) * _WR, :] = jnp.where(mask, er, qe[k * _WR:(k + 1) * _WR, :])


def _ids_body(s_ref, ilo, ihi, qlo, qhi, slo, shi, nlo, nhi, rlo_s, rhi_s):
    rows, cols = qlo.shape
    wrows = ilo.shape[0]
    ce = rows * cols
    wl = wrows * cols
    w0 = s_ref[0]

    rlo_s[...] = _flatroll(ilo[...], jnp.mod(w0, wl))
    rhi_s[...] = _flatroll(ihi[...], jnp.mod(w0, wl))

    slo[...] = qlo[...]
    shi[...] = qhi[...]
    rlo = rlo_s[...]
    rhi = rhi_s[...]
    for k in range(rows // wrows):
        fi = (lax.broadcasted_iota(jnp.int32, (wrows, cols), 0) * cols
              + lax.broadcasted_iota(jnp.int32, (wrows, cols), 1))
        tt = k * wl + fi - w0
        tt = jnp.where(tt < 0, tt + ce, tt)
        mask = tt < wl
        nlo[k * wrows:(k + 1) * wrows, :] = jnp.where(mask, rlo, qlo[k * wrows:(k + 1) * wrows, :])
        nhi[k * wrows:(k + 1) * wrows, :] = jnp.where(mask, rhi, qhi[k * wrows:(k + 1) * wrows, :])


def kernel(embeddings, item_ids, queue_embeddings, queue_item_ids, ptr):
    C, D = queue_embeddings.shape
    B = embeddings.shape[0]
    p = jnp.asarray(jnp.mod(ptr, C), jnp.int32)

    CE = C * D
    G = CE // (_RB * 128)
    emb2 = embeddings.reshape(_WR, 128)
    qe2 = queue_embeddings.reshape(CE // 128, 128)
    scal = jnp.stack([D * p, jnp.int32(0)])

    se2, ne2 = pl.pallas_call(
        _emb_body,
        grid_spec=pltpu.PrefetchScalarGridSpec(
            num_scalar_prefetch=1,
            grid=(G,),
            in_specs=[
                pl.BlockSpec((_WR, 128), lambda g, pr: (jnp.int32(0), jnp.int32(0))),
                pl.BlockSpec((_RB, 128), lambda g, pr: (g, jnp.int32(0))),
            ],
            out_specs=[
                pl.BlockSpec((_RB, 128), lambda g, pr: (g, jnp.int32(0))),
                pl.BlockSpec((_RB, 128), lambda g, pr: (g, jnp.int32(0))),
            ],
            scratch_shapes=[pltpu.VMEM((_WR, 128), jnp.float32)],
        ),
        out_shape=[
            jax.ShapeDtypeStruct((CE // 128, 128), jnp.float32),
            jax.ShapeDtypeStruct((CE // 128, 128), jnp.float32),
        ],
    )(scal, emb2, qe2)

    ilo = item_ids.astype(jnp.int32).reshape(B // 128, 128)
    ihi = jnp.right_shift(item_ids, 32).astype(jnp.int32).reshape(B // 128, 128)
    qlo = queue_item_ids.astype(jnp.int32).reshape(C // 128, 128)
    qhi = jnp.right_shift(queue_item_ids, 32).astype(jnp.int32).reshape(C // 128, 128)
    scal2 = jnp.stack([p, jnp.int32(0)])

    win_spec = pl.BlockSpec((B // 128, 128), lambda g, pr: (jnp.int32(0), jnp.int32(0)))
    full_spec = pl.BlockSpec((C // 128, 128), lambda g, pr: (jnp.int32(0), jnp.int32(0)))
    slo, shi, nlo, nhi = pl.pallas_call(
        _ids_body,
        grid_spec=pltpu.PrefetchScalarGridSpec(
            num_scalar_prefetch=1,
            grid=(1,),
            in_specs=[win_spec, win_spec, full_spec, full_spec],
            out_specs=[full_spec, full_spec, full_spec, full_spec],
            scratch_shapes=[
                pltpu.VMEM((B // 128, 128), jnp.int32),
                pltpu.VMEM((B // 128, 128), jnp.int32),
            ],
        ),
        out_shape=[
            jax.ShapeDtypeStruct((C // 128, 128), jnp.int32),
            jax.ShapeDtypeStruct((C // 128, 128), jnp.int32),
            jax.ShapeDtypeStruct((C // 128, 128), jnp.int32),
            jax.ShapeDtypeStruct((C // 128, 128), jnp.int32),
        ],
    )(scal2, ilo, ihi, qlo, qhi)

    def _to64(hi, lo):
        return (jnp.left_shift(hi.reshape(C).astype(jnp.int64), 32)
                | (lo.reshape(C).astype(jnp.int64) & jnp.int64(0xFFFFFFFF)))

    se = se2.reshape(C, D)
    ne = ne2.reshape(C, D)
    return (se, _to64(shi, slo), ne, _to64(nhi, nlo))
